# Initial kernel scaffold; baseline (speedup 1.0000x reference)
import jax
import jax.numpy as jnp
from jax.experimental import pallas as pl


def kernel(seq_len, rel_embeddings):
    raise NotImplementedError("write your pallas kernel here")


# speedup vs baseline: 9.8414x; 9.8414x over previous
import functools

import jax
import jax.numpy as jnp
from jax import lax
from jax.experimental import pallas as pl
from jax.experimental.pallas import tpu as pltpu
from jax.experimental.pallas import tpu_sc as plsc

MAX_REL_DIST = 128
EMBED_DIM = 16
SEQ_LEN = 2048
TABLE_ROWS = 2 * MAX_REL_DIST + 1
BAND_ROWS = 2 * SEQ_LEN - 1

NUM_CORES = 2
NUM_SUBCORES = 16
NUM_WORKERS = NUM_CORES * NUM_SUBCORES
ROWS_PER_WORKER = SEQ_LEN // NUM_WORKERS


def _band_body(table_hbm, out_hbm, table_v, band_v, sem):
    wid = lax.axis_index("s") * NUM_CORES + lax.axis_index("c")
    base = wid * ROWS_PER_WORKER

    pltpu.sync_copy(table_hbm, table_v)

    lo = (SEQ_LEN - 1) - (base + ROWS_PER_WORKER - 1)
    hi = (2 * SEQ_LEN - 1) - base

    def build(m, carry):
        d = (SEQ_LEN - 1) - m
        idx = jnp.clip(d, -MAX_REL_DIST, MAX_REL_DIST) + MAX_REL_DIST
        band_v[m, :] = table_v[idx, :]
        return carry

    lax.fori_loop(lo, hi, build, 0)

    def fire(r, carry):
        i = base + r
        src = band_v.at[pl.ds((SEQ_LEN - 1) - i, SEQ_LEN), :]
        pltpu.make_async_copy(src, out_hbm.at[i], sem).start()
        return carry

    lax.fori_loop(0, ROWS_PER_WORKER, fire, 0)

    def drain(r, carry):
        src = band_v.at[pl.ds(0, SEQ_LEN), :]
        pltpu.make_async_copy(src, out_hbm.at[base + r], sem).wait()
        return carry

    lax.fori_loop(0, ROWS_PER_WORKER, drain, 0)


_band_call = functools.partial(
    pl.kernel,
    out_type=jax.ShapeDtypeStruct((SEQ_LEN, SEQ_LEN, EMBED_DIM), jnp.float32),
    mesh=plsc.VectorSubcoreMesh(core_axis_name="c", subcore_axis_name="s"),
    compiler_params=pltpu.CompilerParams(use_tc_tiling_on_sc=False),
    scratch_types=[
        pltpu.VMEM((TABLE_ROWS, EMBED_DIM), jnp.float32),
        pltpu.VMEM((BAND_ROWS, EMBED_DIM), jnp.float32),
        pltpu.SemaphoreType.DMA,
    ],
)(_band_body)


def kernel(seq_len, rel_embeddings):
    del seq_len
    return _band_call(rel_embeddings.astype(jnp.float32))


# speedup vs baseline: 9.8541x; 1.0013x over previous
import functools

import jax
import jax.numpy as jnp
from jax import lax
from jax.experimental import pallas as pl
from jax.experimental.pallas import tpu as pltpu
from jax.experimental.pallas import tpu_sc as plsc

MAX_REL_DIST = 128
EMBED_DIM = 16
SEQ_LEN = 2048
TABLE_ROWS = 2 * MAX_REL_DIST + 1
BAND_ROWS = 2 * SEQ_LEN - 1
ROW_WORDS = SEQ_LEN * EMBED_DIM

NUM_CORES = 2
NUM_SUBCORES = 16
NUM_WORKERS = NUM_CORES * NUM_SUBCORES
ROWS_PER_WORKER = SEQ_LEN // NUM_WORKERS


def _band_body(table_hbm, out_hbm, table_v, band_v, sem):
    wid = lax.axis_index("s") * NUM_CORES + lax.axis_index("c")
    base = wid * ROWS_PER_WORKER

    pltpu.sync_copy(table_hbm, table_v)

    lo = (SEQ_LEN - 1) - (base + ROWS_PER_WORKER - 1)
    hi = (2 * SEQ_LEN - 1) - base

    def build(m, carry):
        d = (SEQ_LEN - 1) - m
        idx = jnp.clip(d, -MAX_REL_DIST, MAX_REL_DIST) + MAX_REL_DIST
        band_v[pl.ds(m * EMBED_DIM, EMBED_DIM)] = table_v[pl.ds(idx * EMBED_DIM, EMBED_DIM)]
        return carry

    lax.fori_loop(lo, hi, build, 0)

    def fire(r, carry):
        i = base + r
        src = band_v.at[pl.ds(((SEQ_LEN - 1) - i) * EMBED_DIM, ROW_WORDS)]
        pltpu.make_async_copy(src, out_hbm.at[pl.ds(i * ROW_WORDS, ROW_WORDS)], sem).start()
        return carry

    lax.fori_loop(0, ROWS_PER_WORKER, fire, 0)

    def drain(r, carry):
        src = band_v.at[pl.ds(0, ROW_WORDS)]
        dst = out_hbm.at[pl.ds((base + r) * ROW_WORDS, ROW_WORDS)]
        pltpu.make_async_copy(src, dst, sem).wait()
        return carry

    lax.fori_loop(0, ROWS_PER_WORKER, drain, 0)


_band_call = functools.partial(
    pl.kernel,
    out_type=jax.ShapeDtypeStruct((SEQ_LEN * SEQ_LEN * EMBED_DIM,), jnp.float32),
    mesh=plsc.VectorSubcoreMesh(core_axis_name="c", subcore_axis_name="s"),
    compiler_params=pltpu.CompilerParams(use_tc_tiling_on_sc=False),
    scratch_types=[
        pltpu.VMEM((TABLE_ROWS * EMBED_DIM,), jnp.float32),
        pltpu.VMEM((BAND_ROWS * EMBED_DIM,), jnp.float32),
        pltpu.SemaphoreType.DMA,
    ],
)(_band_body)


def kernel(seq_len, rel_embeddings):
    del seq_len
    flat_table = rel_embeddings.astype(jnp.float32).reshape(-1)
    out_flat = _band_call(flat_table)
    return out_flat.reshape(SEQ_LEN, SEQ_LEN, EMBED_DIM)


# speedup vs baseline: 128.0204x; 12.9916x over previous
import functools

import jax
import jax.numpy as jnp
from jax import lax
from jax.experimental import pallas as pl
from jax.experimental.pallas import tpu as pltpu
from jax.experimental.pallas import tpu_sc as plsc

MAX_REL_DIST = 128
EMBED_DIM = 16
SEQ_LEN = 2048
TABLE_ROWS = 2 * MAX_REL_DIST + 1
BAND_COLS = 4096

NUM_CORES = 2
NUM_SUBCORES = 16
NUM_WORKERS = NUM_CORES * NUM_SUBCORES
ROWS_PER_WORKER = SEQ_LEN // NUM_WORKERS
C_TILES = EMBED_DIM // 8
J_TILES = SEQ_LEN // 128


def _band_body(table_hbm, out_hbm, table_v, bandt_v, sem):
    wid = lax.axis_index("s") * NUM_CORES + lax.axis_index("c")

    pltpu.sync_copy(table_hbm, table_v)

    off_min = (NUM_WORKERS - 1) - wid

    lane = lax.iota(jnp.int32, 16)

    def build(nb, carry):
        m = nb * 16 + lane + off_min
        d = (SEQ_LEN - 1) - m
        idx = jnp.maximum(jnp.minimum(d, MAX_REL_DIST), -MAX_REL_DIST) + MAX_REL_DIST
        for c in range(EMBED_DIM):
            row = plsc.load_gather(table_v, [idx * EMBED_DIM + c])
            bandt_v[c, pl.ds(nb * 16, 16)] = row
        return carry

    lax.fori_loop(0, (SEQ_LEN + 2016) // 16, build, 0)

    def fire(r, carry):
        i = wid + NUM_WORKERS * r

        def fire_jt(jt, carry2):
            start = 8 * (252 - 4 * r + 16 * jt)
            for ct in range(C_TILES):
                src = bandt_v.at[pl.ds(ct * 8, 8), pl.ds(start, 128)]
                pltpu.make_async_copy(src, out_hbm.at[i, ct, jt], sem).start()
            return carry2

        lax.fori_loop(0, J_TILES, fire_jt, 0)
        return carry

    lax.fori_loop(0, ROWS_PER_WORKER, fire, 0)

    def drain(r, carry):
        src = bandt_v.at[pl.ds(0, 8), pl.ds(0, 128)]

        def drain_jt(jt, carry2):
            for ct in range(C_TILES):
                i = wid + NUM_WORKERS * r
                pltpu.make_async_copy(src, out_hbm.at[i, ct, jt], sem).wait()
            return carry2

        lax.fori_loop(0, J_TILES, drain_jt, 0)
        return carry

    lax.fori_loop(0, ROWS_PER_WORKER, drain, 0)


_band_call = functools.partial(
    pl.kernel,
    out_type=jax.ShapeDtypeStruct((SEQ_LEN, C_TILES, J_TILES, 8, 128), jnp.float32),
    mesh=plsc.VectorSubcoreMesh(core_axis_name="c", subcore_axis_name="s"),
    compiler_params=pltpu.CompilerParams(use_tc_tiling_on_sc=False, needs_layout_passes=False),
    scratch_types=[
        pltpu.VMEM((TABLE_ROWS * EMBED_DIM,), jnp.float32),
        pltpu.VMEM((EMBED_DIM, BAND_COLS), jnp.float32),
        pltpu.SemaphoreType.DMA,
    ],
)(_band_body)


def kernel(seq_len, rel_embeddings):
    del seq_len
    flat_table = rel_embeddings.astype(jnp.float32).reshape(-1)
    out5 = _band_call(flat_table)
    out = out5.transpose(0, 1, 3, 2, 4).reshape(SEQ_LEN, EMBED_DIM, SEQ_LEN)
    return out.transpose(0, 2, 1)


# speedup vs baseline: 131.1934x; 1.0248x over previous
import functools

import jax
import jax.numpy as jnp
from jax import lax
from jax.experimental import pallas as pl
from jax.experimental.pallas import tpu as pltpu
from jax.experimental.pallas import tpu_sc as plsc

MAX_REL_DIST = 128
EMBED_DIM = 16
SEQ_LEN = 2048
TABLE_ROWS = 2 * MAX_REL_DIST + 1
BAND_COLS = 4096

NUM_CORES = 2
NUM_SUBCORES = 16
NUM_WORKERS = NUM_CORES * NUM_SUBCORES
ROWS_PER_WORKER = SEQ_LEN // NUM_WORKERS
C_TILES = EMBED_DIM // 8
J_TILES = SEQ_LEN // 128


def _band_body(table_hbm, out_hbm, table_v, bandt_v, sem):
    wid = lax.axis_index("s") * NUM_CORES + lax.axis_index("c")

    pltpu.sync_copy(table_hbm, table_v)

    off_min = (NUM_WORKERS - 1) - wid

    lane = lax.iota(jnp.int32, 16)

    def build(nb, carry):
        m = nb * 16 + lane + off_min
        d = (SEQ_LEN - 1) - m
        idx = jnp.maximum(jnp.minimum(d, MAX_REL_DIST), -MAX_REL_DIST) + MAX_REL_DIST
        for c in range(EMBED_DIM):
            row = plsc.load_gather(table_v, [idx * EMBED_DIM + c])
            bandt_v[c // 8, c % 8, pl.ds(nb * 16, 16)] = row
        return carry

    lax.fori_loop(0, (SEQ_LEN + 2016) // 16, build, 0)

    def fire(r, carry):
        i = wid + NUM_WORKERS * r

        def fire_jt(jt, carry2):
            start = 8 * (252 - 4 * r + 16 * jt)
            src = bandt_v.at[:, :, pl.ds(start, 128)]
            pltpu.make_async_copy(src, out_hbm.at[i, :, jt], sem).start()
            return carry2

        lax.fori_loop(0, J_TILES, fire_jt, 0)
        return carry

    lax.fori_loop(0, ROWS_PER_WORKER, fire, 0)

    def drain(r, carry):
        src = bandt_v.at[:, :, pl.ds(0, 128)]

        def drain_jt(jt, carry2):
            i = wid + NUM_WORKERS * r
            pltpu.make_async_copy(src, out_hbm.at[i, :, jt], sem).wait()
            return carry2

        lax.fori_loop(0, J_TILES, drain_jt, 0)
        return carry

    lax.fori_loop(0, ROWS_PER_WORKER, drain, 0)


_band_call = functools.partial(
    pl.kernel,
    out_type=jax.ShapeDtypeStruct((SEQ_LEN, C_TILES, J_TILES, 8, 128), jnp.float32),
    mesh=plsc.VectorSubcoreMesh(core_axis_name="c", subcore_axis_name="s"),
    compiler_params=pltpu.CompilerParams(use_tc_tiling_on_sc=False, needs_layout_passes=False),
    scratch_types=[
        pltpu.VMEM((TABLE_ROWS * EMBED_DIM,), jnp.float32),
        pltpu.VMEM((C_TILES, 8, BAND_COLS), jnp.float32),
        pltpu.SemaphoreType.DMA,
    ],
)(_band_body)


def kernel(seq_len, rel_embeddings):
    del seq_len
    flat_table = rel_embeddings.astype(jnp.float32).reshape(-1)
    out5 = _band_call(flat_table)
    out = out5.transpose(0, 1, 3, 2, 4).reshape(SEQ_LEN, EMBED_DIM, SEQ_LEN)
    return out.transpose(0, 2, 1)


# speedup vs baseline: 141.0872x; 1.0754x over previous
import functools

import jax
import jax.numpy as jnp
from jax import lax
from jax.experimental import pallas as pl
from jax.experimental.pallas import tpu as pltpu
from jax.experimental.pallas import tpu_sc as plsc

MAX_REL_DIST = 128
EMBED_DIM = 16
SEQ_LEN = 2048
TABLE_ROWS = 2 * MAX_REL_DIST + 1
BAND_COLS = 4096

NUM_CORES = 2
NUM_SUBCORES = 16
NUM_WORKERS = NUM_CORES * NUM_SUBCORES
ROWS_PER_WORKER = SEQ_LEN // NUM_WORKERS
C_TILES = EMBED_DIM // 8
J_TILES = SEQ_LEN // 128


def _band_body(table_hbm, out_hbm, table_v, bandt_v, sem):
    wid = lax.axis_index("s") * NUM_CORES + lax.axis_index("c")

    pltpu.sync_copy(table_hbm, table_v)

    off_min = (NUM_WORKERS - 1) - wid

    lane = lax.iota(jnp.int32, 16)

    def build(nb):
        m = nb * 16 + lane + off_min
        d = (SEQ_LEN - 1) - m
        idx = jnp.maximum(jnp.minimum(d, MAX_REL_DIST), -MAX_REL_DIST) + MAX_REL_DIST
        for c in range(EMBED_DIM):
            row = plsc.load_gather(table_v, [idx * EMBED_DIM + c])
            bandt_v[c // 8, c % 8, pl.ds(nb * 16, 16)] = row

    def fire(r):
        i = wid + NUM_WORKERS * r

        def fire_jt(jt, carry2):
            start = 8 * (252 - 4 * r + 16 * jt)
            src = bandt_v.at[:, :, pl.ds(start, 128)]
            pltpu.make_async_copy(src, out_hbm.at[i, :, jt], sem).start()
            return carry2

        lax.fori_loop(0, J_TILES, fire_jt, 0)

    def build_prefix(nb, carry):
        build(nb)
        return carry

    lax.fori_loop(0, 128, build_prefix, 0)
    fire(ROWS_PER_WORKER - 1)

    def step(k, carry):
        build(126 + 2 * k)
        build(127 + 2 * k)
        fire((ROWS_PER_WORKER - 1) - k)
        return carry

    lax.fori_loop(1, ROWS_PER_WORKER, step, 0)

    def drain(r, carry):
        src = bandt_v.at[:, :, pl.ds(0, 128)]

        def drain_jt(jt, carry2):
            i = wid + NUM_WORKERS * r
            pltpu.make_async_copy(src, out_hbm.at[i, :, jt], sem).wait()
            return carry2

        lax.fori_loop(0, J_TILES, drain_jt, 0)
        return carry

    lax.fori_loop(0, ROWS_PER_WORKER, drain, 0)


_band_call = functools.partial(
    pl.kernel,
    out_type=jax.ShapeDtypeStruct((SEQ_LEN, C_TILES, J_TILES, 8, 128), jnp.float32),
    mesh=plsc.VectorSubcoreMesh(core_axis_name="c", subcore_axis_name="s"),
    compiler_params=pltpu.CompilerParams(use_tc_tiling_on_sc=False, needs_layout_passes=False),
    scratch_types=[
        pltpu.VMEM((TABLE_ROWS * EMBED_DIM,), jnp.float32),
        pltpu.VMEM((C_TILES, 8, BAND_COLS), jnp.float32),
        pltpu.SemaphoreType.DMA,
    ],
)(_band_body)


def kernel(seq_len, rel_embeddings):
    del seq_len
    flat_table = rel_embeddings.astype(jnp.float32).reshape(-1)
    out5 = _band_call(flat_table)
    out = out5.transpose(0, 1, 3, 2, 4).reshape(SEQ_LEN, EMBED_DIM, SEQ_LEN)
    return out.transpose(0, 2, 1)


# speedup vs baseline: 142.7904x; 1.0121x over previous
import functools

import jax
import jax.numpy as jnp
from jax import lax
from jax.experimental import pallas as pl
from jax.experimental.pallas import tpu as pltpu
from jax.experimental.pallas import tpu_sc as plsc

MAX_REL_DIST = 128
EMBED_DIM = 16
SEQ_LEN = 2048
TABLE_ROWS = 2 * MAX_REL_DIST + 1
BAND_COLS = 4096

NUM_CORES = 2
NUM_SUBCORES = 16
NUM_WORKERS = NUM_CORES * NUM_SUBCORES
ROWS_PER_WORKER = SEQ_LEN // NUM_WORKERS
C_TILES = EMBED_DIM // 8
J_TILES = SEQ_LEN // 128


def _band_body(table_hbm, out_hbm, table_v, bandt_v, sem):
    wid = lax.axis_index("s") * NUM_CORES + lax.axis_index("c")

    pltpu.sync_copy(table_hbm, table_v)

    off_min = (NUM_WORKERS - 1) - wid

    lane = lax.iota(jnp.int32, 16)

    def build(nb):
        m = nb * 16 + lane + off_min
        d = (SEQ_LEN - 1) - m
        idx = jnp.maximum(jnp.minimum(d, MAX_REL_DIST), -MAX_REL_DIST) + MAX_REL_DIST
        for c in range(EMBED_DIM):
            row = plsc.load_gather(table_v, [idx * EMBED_DIM + c])
            bandt_v[c // 8, c % 8, pl.ds(nb * 16, 16)] = row

    def fire(r):
        i = wid + NUM_WORKERS * r

        def fire_jt(jt, carry2):
            start = 8 * (252 - 4 * r + 16 * jt)
            src = bandt_v.at[:, :, pl.ds(start, 128)]
            pltpu.make_async_copy(src, out_hbm.at[i, :, jt], sem).start()
            return carry2

        lax.fori_loop(0, J_TILES, fire_jt, 0)

    i63 = wid + NUM_WORKERS * (ROWS_PER_WORKER - 1)

    def prefix(jb, carry):
        for b in range(8):
            build(8 * jb + b)
        src = bandt_v.at[:, :, pl.ds(jb * 128, 128)]
        pltpu.make_async_copy(src, out_hbm.at[i63, :, jb], sem).start()
        return carry

    lax.fori_loop(0, J_TILES, prefix, 0)

    def step(k, carry):
        build(126 + 2 * k)
        build(127 + 2 * k)
        fire((ROWS_PER_WORKER - 1) - k)
        return carry

    lax.fori_loop(1, ROWS_PER_WORKER, step, 0)

    def drain(r, carry):
        src = bandt_v.at[:, :, pl.ds(0, 128)]

        def drain_jt(jt, carry2):
            i = wid + NUM_WORKERS * r
            pltpu.make_async_copy(src, out_hbm.at[i, :, jt], sem).wait()
            return carry2

        lax.fori_loop(0, J_TILES, drain_jt, 0)
        return carry

    lax.fori_loop(0, ROWS_PER_WORKER, drain, 0)


_band_call = functools.partial(
    pl.kernel,
    out_type=jax.ShapeDtypeStruct((SEQ_LEN, C_TILES, J_TILES, 8, 128), jnp.float32),
    mesh=plsc.VectorSubcoreMesh(core_axis_name="c", subcore_axis_name="s"),
    compiler_params=pltpu.CompilerParams(use_tc_tiling_on_sc=False, needs_layout_passes=False),
    scratch_types=[
        pltpu.VMEM((TABLE_ROWS * EMBED_DIM,), jnp.float32),
        pltpu.VMEM((C_TILES, 8, BAND_COLS), jnp.float32),
        pltpu.SemaphoreType.DMA,
    ],
)(_band_body)


def kernel(seq_len, rel_embeddings):
    del seq_len
    flat_table = rel_embeddings.astype(jnp.float32).reshape(-1)
    out5 = _band_call(flat_table)
    out = out5.transpose(0, 1, 3, 2, 4).reshape(SEQ_LEN, EMBED_DIM, SEQ_LEN)
    return out.transpose(0, 2, 1)
